# baseline (speedup 1.0000x reference)
import jax
import jax.numpy as jnp
from jax import lax
from jax.experimental import pallas as pl
from jax.experimental.pallas import tpu as pltpu
from jax.experimental.pallas import tpu_sc as plsc

D_MODEL = 1024
MAX_LEN = 2048

NC = 2
NS = 16
NW = NC * NS

BATCH = 4
SEQ = 2048
B = BATCH * SEQ
B_PER_W = B // NW
W_PER_ROW = SEQ // B_PER_W
CHUNK = 32
N_CHUNKS = B_PER_W // CHUNK
NBUF = 2


def _pe_gather_body(pe_hbm, x_hbm, out_hbm, idx_v, buf0, buf1, sg0, sg1,
                    ss0, ss1):
    bufs = (buf0, buf1)
    sgs = (sg0, sg1)
    sss = (ss0, ss1)
    wid = lax.axis_index("s") * NC + lax.axis_index("c")
    b = wid // W_PER_ROW
    off = (wid % W_PER_ROW) * B_PER_W
    pltpu.sync_copy(x_hbm.at[b, pl.ds(off, B_PER_W)], idx_v)

    def gather(c, s):
        src = pe_hbm.at[idx_v.at[pl.ds(pl.multiple_of(c * CHUNK, CHUNK), CHUNK)]]
        return pltpu.make_async_copy(src, bufs[s], sgs[s])

    def scatter(c, s):
        dst = out_hbm.at[b, pl.ds(pl.multiple_of(off + c * CHUNK, CHUNK), CHUNK)]
        return pltpu.make_async_copy(bufs[s], dst, sss[s])

    for s in range(NBUF):
        gather(s, s).start()

    def step(k, carry):
        for s in range(NBUF):
            c = k * NBUF + s
            gather(c, s).wait()
            scatter(c, s).start()
            scatter(c, s).wait()
            gather(c + NBUF, s).start()
        return carry

    lax.fori_loop(0, (N_CHUNKS - NBUF) // NBUF, step, 0, unroll=False)

    for s in range(NBUF):
        c = N_CHUNKS - NBUF + s
        gather(c, s).wait()
        scatter(c, s).start()
    for s in range(NBUF):
        scatter(N_CHUNKS - NBUF + s, s).wait()


@jax.jit
def kernel(x, pe):
    mesh = plsc.VectorSubcoreMesh(core_axis_name="c", subcore_axis_name="s")
    run = pl.kernel(
        _pe_gather_body,
        mesh=mesh,
        out_type=jax.ShapeDtypeStruct((BATCH, SEQ, 1, D_MODEL), jnp.float32),
        scratch_types=[
            pltpu.VMEM((B_PER_W,), jnp.int32),
            pltpu.VMEM((CHUNK, 1, D_MODEL), jnp.float32),
            pltpu.VMEM((CHUNK, 1, D_MODEL), jnp.float32),
            pltpu.SemaphoreType.DMA,
            pltpu.SemaphoreType.DMA,
            pltpu.SemaphoreType.DMA,
            pltpu.SemaphoreType.DMA,
        ],
    )
    return run(pe, x.astype(jnp.int32))
